# Initial kernel scaffold; baseline (speedup 1.0000x reference)
import jax
import jax.numpy as jnp
from jax.experimental import pallas as pl


def kernel(x, sentence_embedding, W1, a_src1, a_dst1, b1, W2, a_src2, a_dst2, b2, edge_index):
    raise NotImplementedError("write your pallas kernel here")


# speedup vs baseline: 31.9490x; 31.9490x over previous
import jax
import jax.numpy as jnp
from jax import lax
from jax.experimental import pallas as pl
from jax.experimental.pallas import tpu as pltpu
from jax.experimental.pallas import tpu_sc as plsc

N = 10000
E = 320000
D_SENT = 768
HID = 128
NCORES = 2
NSUB = 16
NTILES = NCORES * NSUB
EPT = E // NTILES
BATCH = 128
CH = 4
NB = 80
PAD = NB * BATCH - EPT
ROWS_PT = N // NSUB
NS = 10112
S_PT = NS // NSUB
BN = 1000


def _sc_edge_body(h_hbm, asrc_hbm, adst_hbm, srcp_hbm, dstp_hbm,
                  out_hbm, s_hbm,
                  srcw, dstw, rowbuf, exbuf, asw, adw, acc_sh, s_sh,
                  sem, sem2):
    cid = lax.axis_index("c")
    sid = lax.axis_index("s")
    wid = cid * NSUB + sid

    z16 = jnp.zeros((16,), jnp.float32)
    lane = lax.iota(jnp.int32, 16)

    def _zrow(r, c2):
        for c in range(HID // 16):
            rowbuf[r, pl.ds(c * 16, 16)] = z16
        return c2
    lax.fori_loop(0, BATCH, _zrow, 0)
    for c in range(BATCH // 16):
        exbuf[pl.ds(c * 16, 16)] = z16
    base = sid * ROWS_PT
    off = 0
    for sz in (128, 128, 128, 128, ROWS_PT - 512):
        pltpu.sync_copy(rowbuf.at[pl.ds(0, sz)],
                        acc_sh.at[pl.ds(base + off, sz)])
        off += sz
    sbase = sid * S_PT
    soff = 0
    for sz in (128, 128, 128, 128, S_PT - 512):
        pltpu.sync_copy(exbuf.at[pl.ds(0, sz)],
                        s_sh.at[pl.ds(sbase + soff, sz)])
        soff += sz
    plsc.subcore_barrier()

    def _window(w, c1):
        pltpu.sync_copy(srcp_hbm.at[wid, pl.ds(w * CH, CH)], srcw)
        pltpu.sync_copy(dstp_hbm.at[wid, pl.ds(w * CH, CH)], dstw)

        def _batch(jj, c2):
            j = w * CH + jj
            cp1 = pltpu.async_copy(h_hbm.at[srcw.at[jj]], rowbuf, sem)
            cp2 = pltpu.async_copy(asrc_hbm.at[srcw.at[jj]], asw, sem2)
            cp3 = pltpu.async_copy(adst_hbm.at[dstw.at[jj]], adw, sem2)
            cp2.wait()
            cp3.wait()
            exs = []
            for k in range(BATCH // 16):
                e = asw[pl.ds(k * 16, 16)] + adw[pl.ds(k * 16, 16)]
                e = jnp.maximum(e, jnp.full((16,), 0.2, jnp.float32) * e)
                ex = jnp.exp(e)
                valid = (jnp.full((16,), 0, jnp.int32) + j * BATCH
                         + k * 16 + lane) < EPT
                ex = jnp.where(valid, ex, z16)
                exbuf[pl.ds(k * 16, 16)] = ex
                exs.append(ex)
            cp1.wait()
            for k in range(BATCH // 16):
                exk = exs[k]

                def _row(l, c3, exk=exk, k=k):
                    r = k * 16 + l
                    spl = jnp.take_along_axis(
                        exk, jnp.full((16,), l, jnp.int32), axis=0,
                        mode="promise_in_bounds")
                    for c in range(HID // 16):
                        rowbuf[r, pl.ds(c * 16, 16)] = (
                            rowbuf[r, pl.ds(c * 16, 16)] * spl)
                    return c3
                lax.fori_loop(0, 16, _row, 0)
            pltpu.sync_copy(rowbuf, acc_sh.at[dstw.at[jj]], add=True)
            pltpu.sync_copy(exbuf, s_sh.at[dstw.at[jj]], add=True)
            return c2
        lax.fori_loop(0, CH, _batch, 0)
        return c1
    lax.fori_loop(0, NB // CH, _window, 0)

    plsc.subcore_barrier()
    pltpu.sync_copy(acc_sh.at[pl.ds(base, ROWS_PT)],
                    out_hbm.at[cid, pl.ds(base, ROWS_PT)])
    pltpu.sync_copy(s_sh.at[pl.ds(sbase, S_PT)],
                    s_hbm.at[cid, pl.ds(sbase, S_PT)])


_sc_edge = pl.kernel(
    _sc_edge_body,
    out_type=[jax.ShapeDtypeStruct((NCORES, N, HID), jnp.float32),
              jax.ShapeDtypeStruct((NCORES, NS), jnp.float32)],
    mesh=plsc.VectorSubcoreMesh(core_axis_name="c", subcore_axis_name="s"),
    scratch_types=[
        pltpu.VMEM((CH, BATCH), jnp.int32),
        pltpu.VMEM((CH, BATCH), jnp.int32),
        pltpu.VMEM((BATCH, HID), jnp.float32),
        pltpu.VMEM((BATCH,), jnp.float32),
        pltpu.VMEM((BATCH,), jnp.float32),
        pltpu.VMEM((BATCH,), jnp.float32),
        pltpu.VMEM_SHARED((N, HID), jnp.float32),
        pltpu.VMEM_SHARED((NS,), jnp.float32),
        pltpu.SemaphoreType.DMA,
        pltpu.SemaphoreType.DMA,
    ],
    compiler_params=pltpu.CompilerParams(use_tc_tiling_on_sc=False),
)


def _tc1_body(x_ref, sent_ref, w1_ref, as_w_ref, ad_w_ref,
              h_ref, as_ref, ad_ref):
    w1 = w1_ref[...]
    c1 = jnp.dot(sent_ref[...], w1[D_SENT:], preferred_element_type=jnp.float32)
    h = jnp.dot(x_ref[...], w1[:D_SENT], preferred_element_type=jnp.float32) + c1
    h_ref[...] = h
    as_ref[...] = jnp.dot(h, as_w_ref[...], preferred_element_type=jnp.float32)
    ad_ref[...] = jnp.dot(h, ad_w_ref[...], preferred_element_type=jnp.float32)


_tc1 = pl.pallas_call(
    _tc1_body,
    grid=(N // BN,),
    in_specs=[
        pl.BlockSpec((BN, D_SENT), lambda i: (i, 0)),
        pl.BlockSpec((1, D_SENT), lambda i: (0, 0)),
        pl.BlockSpec((2 * D_SENT, HID), lambda i: (0, 0)),
        pl.BlockSpec((HID, 1), lambda i: (0, 0)),
        pl.BlockSpec((HID, 1), lambda i: (0, 0)),
    ],
    out_specs=[
        pl.BlockSpec((BN, HID), lambda i: (i, 0)),
        pl.BlockSpec((BN, 1), lambda i: (i, 0)),
        pl.BlockSpec((BN, 1), lambda i: (i, 0)),
    ],
    out_shape=[
        jax.ShapeDtypeStruct((N, HID), jnp.float32),
        jax.ShapeDtypeStruct((N, 1), jnp.float32),
        jax.ShapeDtypeStruct((N, 1), jnp.float32),
    ],
)


def _tc2_body(p_ref, s_ref, b1_ref, w2_ref, as_w_ref, ad_w_ref,
              h_ref, as_ref, ad_ref):
    s = s_ref[0] + s_ref[1]
    g = (p_ref[0] + p_ref[1]) / (s + 1e-16) + b1_ref[...]
    g = jnp.where(g > 0, g, jnp.exp(g) - 1.0)
    h = jnp.dot(g, w2_ref[...], preferred_element_type=jnp.float32)
    h_ref[...] = h
    as_ref[...] = jnp.dot(h, as_w_ref[...], preferred_element_type=jnp.float32)
    ad_ref[...] = jnp.dot(h, ad_w_ref[...], preferred_element_type=jnp.float32)


_tc2 = pl.pallas_call(
    _tc2_body,
    grid=(N // BN,),
    in_specs=[
        pl.BlockSpec((NCORES, BN, HID), lambda i: (0, i, 0)),
        pl.BlockSpec((NCORES, BN, 1), lambda i: (0, i, 0)),
        pl.BlockSpec((1, HID), lambda i: (0, 0)),
        pl.BlockSpec((HID, HID), lambda i: (0, 0)),
        pl.BlockSpec((HID, 1), lambda i: (0, 0)),
        pl.BlockSpec((HID, 1), lambda i: (0, 0)),
    ],
    out_specs=[
        pl.BlockSpec((BN, HID), lambda i: (i, 0)),
        pl.BlockSpec((BN, 1), lambda i: (i, 0)),
        pl.BlockSpec((BN, 1), lambda i: (i, 0)),
    ],
    out_shape=[
        jax.ShapeDtypeStruct((N, HID), jnp.float32),
        jax.ShapeDtypeStruct((N, 1), jnp.float32),
        jax.ShapeDtypeStruct((N, 1), jnp.float32),
    ],
)


def _tc3_body(p_ref, s_ref, b2_ref, out_ref):
    s = s_ref[0] + s_ref[1]
    out_ref[...] = (p_ref[0] + p_ref[1]) / (s + 1e-16) + b2_ref[...]


_tc3 = pl.pallas_call(
    _tc3_body,
    grid=(N // BN,),
    in_specs=[
        pl.BlockSpec((NCORES, BN, HID), lambda i: (0, i, 0)),
        pl.BlockSpec((NCORES, BN, 1), lambda i: (0, i, 0)),
        pl.BlockSpec((1, HID), lambda i: (0, 0)),
    ],
    out_specs=pl.BlockSpec((BN, HID), lambda i: (i, 0)),
    out_shape=jax.ShapeDtypeStruct((N, HID), jnp.float32),
)


def kernel(x, sentence_embedding, W1, a_src1, a_dst1, b1,
           W2, a_src2, a_dst2, b2, edge_index):
    pad = jnp.broadcast_to(jnp.arange(PAD, dtype=jnp.int32), (NTILES, PAD))
    src = jnp.concatenate([edge_index[0].reshape(NTILES, EPT), pad], axis=1)
    dst = jnp.concatenate([edge_index[1].reshape(NTILES, EPT), pad], axis=1)
    srcp = src.reshape(NTILES, NB, BATCH)
    dstp = dst.reshape(NTILES, NB, BATCH)

    sent2 = sentence_embedding.reshape(1, D_SENT)
    h1, as1, ad1 = _tc1(x, sent2, W1,
                        a_src1.reshape(HID, 1), a_dst1.reshape(HID, 1))
    p1, s1 = _sc_edge(h1, as1.reshape(N), ad1.reshape(N), srcp, dstp)
    s1r = s1[:, :N].reshape(NCORES, N, 1)
    h2, as2, ad2 = _tc2(p1, s1r, b1.reshape(1, HID), W2,
                        a_src2.reshape(HID, 1), a_dst2.reshape(HID, 1))
    p2, s2 = _sc_edge(h2, as2.reshape(N), ad2.reshape(N), srcp, dstp)
    s2r = s2[:, :N].reshape(NCORES, N, 1)
    return _tc3(p2, s2r, b2.reshape(1, HID))
